# baseline (speedup 1.0000x reference)
import functools

import jax
import jax.numpy as jnp
from jax import lax
from jax.experimental import pallas as pl
from jax.experimental.pallas import tpu as pltpu
from jax.experimental.pallas import tpu_sc as plsc

_HIGH = jax.lax.Precision.HIGHEST
_K = 3


def _dot(a, b, dims, precision=_HIGH):
    return jax.lax.dot_general(a, b, (dims, ((), ())), precision=precision,
                               preferred_element_type=jnp.float32)


def _pool_kernel(qe_ref, txt_ref, img_ref, pooled_ref, gate_ref, *, s_blk):
    b = pl.program_id(0)
    s = pl.program_id(1)
    ns = pl.num_programs(1)
    w = 1.0 - txt_ref[pl.ds(b, 1), pl.ds(s * s_blk, s_blk)]
    part = _dot(w, qe_ref[0], ((1,), (0,)))

    @pl.when(s == 0)
    def _():
        pooled_ref[0] = part

    @pl.when(s > 0)
    def _():
        pooled_ref[0] += part

    @pl.when(s == ns - 1)
    def _():
        wsum = jnp.sum(1.0 - txt_ref[pl.ds(b, 1), :])
        pooled_ref[0] = pooled_ref[0] / (wsum + 1e-6)
        vu = 1.0 - jnp.mean(img_ref[pl.ds(b, 1), :])
        tu = 1.0 - jnp.mean(txt_ref[pl.ds(b, 1), :])
        gate_ref[0] = jnp.full((1, 128), (1.0 + vu) * (1.0 - 0.5 * tu),
                               jnp.float32)


def _tables_kernel(t_ref, we_ref, wq_ref, bq_ref, be_ref, pooled_ref,
                   dot_ref, tn_ref, qn_ref):
    j = pl.program_id(1)
    qblk = _dot(pooled_ref[...], wq_ref[...], ((1,), (1,)),
                precision=jax.lax.Precision.DEFAULT) + bq_ref[...]
    p = _dot(t_ref[...], we_ref[...], ((1,), (1,)),
             precision=jax.lax.Precision.DEFAULT) + be_ref[...]
    ones = jnp.ones((1, p.shape[1]), jnp.float32)
    tn2_part = _dot(ones, p * p, ((1,), (1,)))
    dot_part = _dot(qblk, p, ((1,), (1,)))
    qn2 = jnp.sum(qblk * qblk, axis=1, keepdims=True)

    @pl.when(j == 0)
    def _():
        dot_ref[...] = dot_part
        tn_ref[...] = tn2_part
        qn_ref[...] = jnp.broadcast_to(qn2, qn_ref.shape)

    @pl.when(j > 0)
    def _():
        dot_ref[...] += dot_part
        tn_ref[...] += tn2_part
        qn_ref[...] += jnp.broadcast_to(qn2, qn_ref.shape)

    @pl.when(j == pl.num_programs(1) - 1)
    def _():
        tn_ref[...] = jnp.sqrt(tn_ref[...])
        qn_ref[...] = jnp.sqrt(qn_ref[...])


def _sc_topk_body(dot_hbm, tn_hbm, ids_hbm, qn_hbm, gate_hbm, table_hbm,
                  emb_hbm, sc_hbm,
                  ids_v, d_v, t_v, s_v, qn_v, gate_v, sc_out_v, tok_v, rows_v,
                  sem, *, nb, e, h):
    wid = lax.axis_index("s") * 2 + lax.axis_index("c")
    nchunk = e // 16
    iota16 = lax.broadcasted_iota(jnp.int32, (16,), 0)

    @pl.when(wid < nb)
    def _():
        b = wid
        pltpu.sync_copy(ids_hbm.at[b], ids_v)
        pltpu.sync_copy(dot_hbm.at[b], d_v)
        pltpu.sync_copy(tn_hbm.at[0], t_v)
        pltpu.sync_copy(qn_hbm.at[b], qn_v)
        pltpu.sync_copy(gate_hbm.at[b], gate_v)
        qnv = qn_v[...]
        gav = gate_v[...]

        def score_chunk(i, _):
            idx = ids_v[pl.ds(i * 16, 16)]
            gd = plsc.load_gather(d_v, [idx])
            gt = plsc.load_gather(t_v, [idx])
            s = gd * gav / jnp.maximum(qnv * gt, 1e-8)
            s_v[pl.ds(i * 16, 16)] = s
            return 0

        lax.fori_loop(0, nchunk, score_chunk, 0, unroll=4)

        toks_row = jnp.zeros((16,), jnp.int32)
        sc_row = jnp.zeros((16,), jnp.float32)
        for k in range(_K):
            def max_chunk(i, m):
                return jnp.maximum(m, s_v[pl.ds(i * 16, 16)])

            m = lax.fori_loop(0, nchunk, max_chunk,
                              jnp.full((16,), -jnp.inf, jnp.float32),
                              unroll=4)
            gmax = jnp.max(m)

            def idx_chunk(i, emin):
                sc = s_v[pl.ds(i * 16, 16)]
                cand = jnp.where(sc >= gmax, iota16 + i * 16,
                                 jnp.int32(2 ** 30))
                return jnp.minimum(emin, jnp.min(cand))

            emin = lax.fori_loop(0, nchunk, idx_chunk, jnp.int32(2 ** 30),
                                 unroll=4)
            ev = jnp.full((16,), 0, jnp.int32) + emin
            tokv = plsc.load_gather(ids_v, [ev])
            toks_row = jnp.where(iota16 == k, tokv, toks_row)
            sc_row = jnp.where(iota16 == k, jnp.full((16,), gmax), sc_row)
            plsc.store_scatter(s_v, [ev],
                               jnp.full((16,), -jnp.inf, jnp.float32),
                               mask=iota16 == 0)

        sc_out_v[...] = sc_row
        pltpu.sync_copy(sc_out_v, sc_hbm.at[b])
        tok_v[...] = toks_row
        pltpu.async_copy(table_hbm.at[tok_v], rows_v, sem).wait()
        pltpu.sync_copy(rows_v.at[pl.ds(0, _K)], emb_hbm.at[b])


def kernel(question_embeddings, pseudo_text, img_conf, txt_conf,
           embedding_table, Wq, bq, We, be):
    nb, s_total, h = question_embeddings.shape
    e = pseudo_text.shape[1]
    v = embedding_table.shape[0]
    vpad = ((v + 127) // 128) * 128
    s_blk = 512
    ho_blk = 512
    wq_blk = 1024

    tpad = jnp.pad(embedding_table, ((0, vpad - v), (0, 0)))
    be2 = be.reshape(1, h)
    bq2 = bq.reshape(1, h)

    pooled, gates = pl.pallas_call(
        functools.partial(_pool_kernel, s_blk=s_blk),
        grid=(nb, s_total // s_blk),
        in_specs=[
            pl.BlockSpec((1, s_blk, h), lambda b, s: (b, s, 0)),
            pl.BlockSpec((nb, s_total), lambda b, s: (0, 0)),
            pl.BlockSpec((nb, s_total), lambda b, s: (0, 0)),
        ],
        out_specs=[
            pl.BlockSpec((1, 1, h), lambda b, s: (b, 0, 0)),
            pl.BlockSpec((1, 1, 128), lambda b, s: (b, 0, 0)),
        ],
        out_shape=[
            jax.ShapeDtypeStruct((nb, 1, h), jnp.float32),
            jax.ShapeDtypeStruct((nb, 1, 128), jnp.float32),
        ],
        compiler_params=pltpu.CompilerParams(
            dimension_semantics=("parallel", "arbitrary")),
    )(question_embeddings, txt_conf, img_conf)
    pooled = pooled.reshape(nb, h)

    v_split = 2
    v_rows = vpad // v_split
    dot, tn, qn_arr = pl.pallas_call(
        _tables_kernel,
        grid=(v_split, h // ho_blk),
        in_specs=[
            pl.BlockSpec((v_rows, h), lambda i, j: (i, 0)),
            pl.BlockSpec((ho_blk, h), lambda i, j: (j, 0)),
            pl.BlockSpec((ho_blk, h), lambda i, j: (j, 0)),
            pl.BlockSpec((1, ho_blk), lambda i, j: (0, j)),
            pl.BlockSpec((1, ho_blk), lambda i, j: (0, j)),
            pl.BlockSpec((nb, h), lambda i, j: (0, 0)),
        ],
        out_specs=[
            pl.BlockSpec((nb, v_rows), lambda i, j: (0, i)),
            pl.BlockSpec((1, v_rows), lambda i, j: (0, i)),
            pl.BlockSpec((nb, 128), lambda i, j: (0, i)),
        ],
        out_shape=[
            jax.ShapeDtypeStruct((nb, vpad), jnp.float32),
            jax.ShapeDtypeStruct((1, vpad), jnp.float32),
            jax.ShapeDtypeStruct((nb, 128 * v_split), jnp.float32),
        ],
        compiler_params=pltpu.CompilerParams(
            dimension_semantics=("parallel", "arbitrary")),
    )(tpad, We, Wq, bq2, be2, pooled)

    qn16 = qn_arr[:, :16]
    gate16 = gates.reshape(nb, 128)[:, :16]

    mesh = plsc.VectorSubcoreMesh(core_axis_name="c", subcore_axis_name="s")
    sc_fn = functools.partial(
        pl.kernel,
        mesh=mesh,
        compiler_params=pltpu.CompilerParams(needs_layout_passes=False),
        out_type=[
            jax.ShapeDtypeStruct((nb, _K, h), jnp.float32),
            jax.ShapeDtypeStruct((nb, 16), jnp.float32),
        ],
        scratch_types=[
            pltpu.VMEM((e,), jnp.int32),
            pltpu.VMEM((vpad,), jnp.float32),
            pltpu.VMEM((vpad,), jnp.float32),
            pltpu.VMEM((e,), jnp.float32),
            pltpu.VMEM((16,), jnp.float32),
            pltpu.VMEM((16,), jnp.float32),
            pltpu.VMEM((16,), jnp.float32),
            pltpu.VMEM((16,), jnp.int32),
            pltpu.VMEM((16, h), jnp.float32),
            pltpu.SemaphoreType.DMA,
        ],
    )(functools.partial(_sc_topk_body, nb=nb, e=e, h=h))
    emb, sc16 = sc_fn(dot, tn, pseudo_text, qn16, gate16, embedding_table)

    return emb[:, :, None, :], sc16[:, :_K]


# speedup vs baseline: 1.0680x; 1.0680x over previous
import functools

import jax
import jax.numpy as jnp
from jax import lax
from jax.experimental import pallas as pl
from jax.experimental.pallas import tpu as pltpu
from jax.experimental.pallas import tpu_sc as plsc

_HIGH = jax.lax.Precision.HIGHEST
_K = 3


def _dot(a, b, dims, precision=_HIGH):
    return jax.lax.dot_general(a, b, (dims, ((), ())), precision=precision,
                               preferred_element_type=jnp.float32)


def _pool_kernel(qe_ref, txt_ref, img_ref, pooled_ref, gate_ref, *, s_blk):
    b = pl.program_id(0)
    s = pl.program_id(1)
    ns = pl.num_programs(1)
    w = 1.0 - txt_ref[pl.ds(b, 1), pl.ds(s * s_blk, s_blk)]
    part = _dot(w, qe_ref[0], ((1,), (0,)))

    @pl.when(s == 0)
    def _():
        pooled_ref[0] = part

    @pl.when(s > 0)
    def _():
        pooled_ref[0] += part

    @pl.when(s == ns - 1)
    def _():
        wsum = jnp.sum(1.0 - txt_ref[pl.ds(b, 1), :])
        pooled_ref[0] = pooled_ref[0] / (wsum + 1e-6)
        vu = 1.0 - jnp.mean(img_ref[pl.ds(b, 1), :])
        tu = 1.0 - jnp.mean(txt_ref[pl.ds(b, 1), :])
        gate_ref[0] = jnp.full((1, 128), (1.0 + vu) * (1.0 - 0.5 * tu),
                               jnp.float32)


def _tables_kernel(t_ref, we_ref, wq_ref, bq_ref, be_ref, pooled_ref,
                   dot_ref, tn2_ref, qn2_ref):
    j = pl.program_id(1)
    qblk = _dot(pooled_ref[...], wq_ref[...], ((1,), (1,)),
                precision=jax.lax.Precision.DEFAULT) + bq_ref[...]
    p = _dot(t_ref[...], we_ref[...], ((1,), (1,)),
             precision=jax.lax.Precision.DEFAULT) + be_ref[...]
    ones = jnp.ones((1, p.shape[1]), jnp.float32)
    tn2_part = _dot(ones, p * p, ((1,), (1,)))
    dot_part = _dot(qblk, p, ((1,), (1,)))
    qn2 = jnp.sum(qblk * qblk, axis=1, keepdims=True)

    @pl.when(j == 0)
    def _():
        dot_ref[0] = dot_part
        tn2_ref[0] = tn2_part
        qn2_ref[0] = jnp.broadcast_to(qn2, qn2_ref.shape[1:])

    @pl.when(j > 0)
    def _():
        dot_ref[0] += dot_part
        tn2_ref[0] += tn2_part
        qn2_ref[0] += jnp.broadcast_to(qn2, qn2_ref.shape[1:])


def _rsqrt16(z):
    yi = jnp.int32(0x5F3759DF) - (plsc.bitcast(z, jnp.int32) >> 1)
    y = plsc.bitcast(yi, jnp.float32)
    for _ in range(3):
        y = y * (1.5 - 0.5 * z * y * y)
    return y


def _sc_topk_body(dot_hbm, tn2_hbm, ids_hbm, qn2_hbm, gate_hbm, table_hbm,
                  emb_hbm, sc_hbm,
                  ids_v, d0_v, d1_v, t0_v, t1_v, s_v, qa_v, qb_v, gate_v,
                  sc_out_v, tok_v, rows_v, sem, *, nb, e, h):
    wid = lax.axis_index("s") * 2 + lax.axis_index("c")
    nchunk = e // 16
    iota16 = lax.broadcasted_iota(jnp.int32, (16,), 0)

    @pl.when(wid < nb)
    def _():
        b = wid
        pltpu.sync_copy(ids_hbm.at[b], ids_v)
        pltpu.sync_copy(dot_hbm.at[0, b], d0_v)
        pltpu.sync_copy(dot_hbm.at[1, b], d1_v)
        pltpu.sync_copy(tn2_hbm.at[0, 0], t0_v)
        pltpu.sync_copy(tn2_hbm.at[1, 0], t1_v)
        pltpu.sync_copy(qn2_hbm.at[0, b], qa_v)
        pltpu.sync_copy(qn2_hbm.at[1, b], qb_v)
        pltpu.sync_copy(gate_hbm.at[b], gate_v)
        qn2v = qa_v[pl.ds(0, 16)] + qb_v[pl.ds(0, 16)]
        gav = gate_v[...]

        def score_chunk(i, _):
            idx = ids_v[pl.ds(i * 16, 16)]
            gd = plsc.load_gather(d0_v, [idx]) + plsc.load_gather(d1_v, [idx])
            gt2 = plsc.load_gather(t0_v, [idx]) + plsc.load_gather(t1_v, [idx])
            z = qn2v * gt2
            den = jnp.maximum(z * _rsqrt16(z), 1e-8)
            s_v[pl.ds(i * 16, 16)] = gd * gav / den
            return 0

        lax.fori_loop(0, nchunk, score_chunk, 0, unroll=4)

        toks_row = jnp.zeros((16,), jnp.int32)
        sc_row = jnp.zeros((16,), jnp.float32)
        for k in range(_K):
            def max_chunk(i, m):
                return jnp.maximum(m, s_v[pl.ds(i * 16, 16)])

            m = lax.fori_loop(0, nchunk, max_chunk,
                              jnp.full((16,), -jnp.inf, jnp.float32),
                              unroll=4)
            gmax = jnp.max(m)

            def idx_chunk(i, emin):
                sc = s_v[pl.ds(i * 16, 16)]
                cand = jnp.where(sc >= gmax, iota16 + i * 16,
                                 jnp.int32(2 ** 30))
                return jnp.minimum(emin, jnp.min(cand))

            emin = lax.fori_loop(0, nchunk, idx_chunk, jnp.int32(2 ** 30),
                                 unroll=4)
            ev = jnp.full((16,), 0, jnp.int32) + emin
            tokv = plsc.load_gather(ids_v, [ev])
            toks_row = jnp.where(iota16 == k, tokv, toks_row)
            sc_row = jnp.where(iota16 == k, jnp.full((16,), gmax), sc_row)
            plsc.store_scatter(s_v, [ev],
                               jnp.full((16,), -jnp.inf, jnp.float32),
                               mask=iota16 == 0)

        sc_out_v[...] = sc_row
        pltpu.sync_copy(sc_out_v, sc_hbm.at[b])
        tok_v[...] = toks_row
        pltpu.async_copy(table_hbm.at[tok_v], rows_v, sem).wait()
        pltpu.sync_copy(rows_v.at[pl.ds(0, _K)], emb_hbm.at[b])


def kernel(question_embeddings, pseudo_text, img_conf, txt_conf,
           embedding_table, Wq, bq, We, be):
    nb, s_total, h = question_embeddings.shape
    e = pseudo_text.shape[1]
    v = embedding_table.shape[0]
    vpad = ((v + 127) // 128) * 128
    s_blk = 512
    ho_blk = 512
    h_split = 2
    nj = h // ho_blk // h_split

    tpad = jnp.pad(embedding_table, ((0, vpad - v), (0, 0)))
    be2 = be.reshape(1, h)
    bq2 = bq.reshape(1, h)

    pooled, gates = pl.pallas_call(
        functools.partial(_pool_kernel, s_blk=s_blk),
        grid=(nb, s_total // s_blk),
        in_specs=[
            pl.BlockSpec((1, s_blk, h), lambda b, s: (b, s, 0)),
            pl.BlockSpec((nb, s_total), lambda b, s: (0, 0)),
            pl.BlockSpec((nb, s_total), lambda b, s: (0, 0)),
        ],
        out_specs=[
            pl.BlockSpec((1, 1, h), lambda b, s: (b, 0, 0)),
            pl.BlockSpec((1, 1, 128), lambda b, s: (b, 0, 0)),
        ],
        out_shape=[
            jax.ShapeDtypeStruct((nb, 1, h), jnp.float32),
            jax.ShapeDtypeStruct((nb, 1, 128), jnp.float32),
        ],
        compiler_params=pltpu.CompilerParams(
            dimension_semantics=("parallel", "arbitrary")),
    )(question_embeddings, txt_conf, img_conf)
    pooled = pooled.reshape(nb, h)

    dot2, tn2, qn2 = pl.pallas_call(
        _tables_kernel,
        grid=(h_split, nj),
        in_specs=[
            pl.BlockSpec((vpad, h), lambda i, j: (0, 0)),
            pl.BlockSpec((ho_blk, h), lambda i, j: (i * 4 + j, 0)),
            pl.BlockSpec((ho_blk, h), lambda i, j: (i * 4 + j, 0)),
            pl.BlockSpec((1, ho_blk), lambda i, j: (0, i * 4 + j)),
            pl.BlockSpec((1, ho_blk), lambda i, j: (0, i * 4 + j)),
            pl.BlockSpec((nb, h), lambda i, j: (0, 0)),
        ],
        out_specs=[
            pl.BlockSpec((1, nb, vpad), lambda i, j: (i, 0, 0)),
            pl.BlockSpec((1, 1, vpad), lambda i, j: (i, 0, 0)),
            pl.BlockSpec((1, nb, 128), lambda i, j: (i, 0, 0)),
        ],
        out_shape=[
            jax.ShapeDtypeStruct((h_split, nb, vpad), jnp.float32),
            jax.ShapeDtypeStruct((h_split, 1, vpad), jnp.float32),
            jax.ShapeDtypeStruct((h_split, nb, 128), jnp.float32),
        ],
        compiler_params=pltpu.CompilerParams(
            dimension_semantics=("parallel", "arbitrary")),
    )(tpad, We, Wq, bq2, be2, pooled)

    gate16 = gates.reshape(nb, 128)[:, :16]

    mesh = plsc.VectorSubcoreMesh(core_axis_name="c", subcore_axis_name="s")
    sc_fn = functools.partial(
        pl.kernel,
        mesh=mesh,
        compiler_params=pltpu.CompilerParams(needs_layout_passes=False),
        out_type=[
            jax.ShapeDtypeStruct((nb, _K, h), jnp.float32),
            jax.ShapeDtypeStruct((nb, 16), jnp.float32),
        ],
        scratch_types=[
            pltpu.VMEM((e,), jnp.int32),
            pltpu.VMEM((vpad,), jnp.float32),
            pltpu.VMEM((vpad,), jnp.float32),
            pltpu.VMEM((vpad,), jnp.float32),
            pltpu.VMEM((vpad,), jnp.float32),
            pltpu.VMEM((e,), jnp.float32),
            pltpu.VMEM((128,), jnp.float32),
            pltpu.VMEM((128,), jnp.float32),
            pltpu.VMEM((16,), jnp.float32),
            pltpu.VMEM((16,), jnp.float32),
            pltpu.VMEM((16,), jnp.int32),
            pltpu.VMEM((16, h), jnp.float32),
            pltpu.SemaphoreType.DMA,
        ],
    )(functools.partial(_sc_topk_body, nb=nb, e=e, h=h))
    emb, sc16 = sc_fn(dot2, tn2, pseudo_text, qn2, gate16, embedding_table)

    return emb[:, :, None, :], sc16[:, :_K]
